# baseline (speedup 1.0000x reference)
import functools

import jax
import jax.numpy as jnp
from jax.experimental import pallas as pl
from jax.experimental.pallas import tpu as pltpu

TOKEN_BLOCK = 512


def _router_block_kernel(x_ref, w_ref, logits_ref, aff_ref, idx_ref, *, n_experts):
    x = x_ref[...]
    w = w_ref[...]
    logits = jax.lax.dot_general(
        x, w, (((1,), (1,)), ((), ())), preferred_element_type=jnp.float32
    )
    logits_ref[...] = logits

    m = jnp.max(logits, axis=1, keepdims=True)
    e = jnp.exp(logits - m)
    s = jnp.sum(e, axis=1, keepdims=True)
    aff = e / s
    aff_ref[...] = aff

    lane = jax.lax.broadcasted_iota(jnp.int32, aff.shape, 1)
    m1 = jnp.max(aff, axis=1, keepdims=True)
    i1 = jnp.min(jnp.where(aff == m1, lane, n_experts), axis=1, keepdims=True)
    masked = jnp.where(lane == i1, -jnp.inf, aff)
    m2 = jnp.max(masked, axis=1, keepdims=True)
    i2 = jnp.min(jnp.where(masked == m2, lane, n_experts), axis=1, keepdims=True)
    idx_ref[...] = jnp.concatenate([i1, i2], axis=1)


def kernel(hidden_states, W):
    S, B, H = hidden_states.shape
    E, _ = W.shape
    T = S * B
    x = hidden_states.reshape(T, H)
    tb = TOKEN_BLOCK
    grid = (T // tb,)

    logits, aff, idx = pl.pallas_call(
        functools.partial(_router_block_kernel, n_experts=E),
        grid=grid,
        in_specs=[
            pl.BlockSpec((tb, H), lambda i: (i, 0)),
            pl.BlockSpec((E, H), lambda i: (0, 0)),
        ],
        out_specs=[
            pl.BlockSpec((tb, E), lambda i: (i, 0)),
            pl.BlockSpec((tb, E), lambda i: (i, 0)),
            pl.BlockSpec((tb, 2), lambda i: (i, 0)),
        ],
        out_shape=[
            jax.ShapeDtypeStruct((T, E), jnp.float32),
            jax.ShapeDtypeStruct((T, E), jnp.float32),
            jax.ShapeDtypeStruct((T, 2), jnp.int32),
        ],
        compiler_params=pltpu.CompilerParams(
            dimension_semantics=("parallel",),
        ),
    )(x, W)
    return logits, aff, idx


# speedup vs baseline: 1.0663x; 1.0663x over previous
import functools

import jax
import jax.numpy as jnp
from jax.experimental import pallas as pl
from jax.experimental.pallas import tpu as pltpu

TOKEN_BLOCK = 2048


def _router_block_kernel(x_ref, w_ref, logits_ref, aff_ref, idx_ref, *, n_experts):
    x = x_ref[...]
    w = w_ref[...]
    logits = jax.lax.dot_general(
        x, w, (((1,), (1,)), ((), ())), preferred_element_type=jnp.float32
    )
    logits_ref[...] = logits

    m = jnp.max(logits, axis=1, keepdims=True)
    e = jnp.exp(logits - m)
    s = jnp.sum(e, axis=1, keepdims=True)
    aff = e / s
    aff_ref[...] = aff

    lane = jax.lax.broadcasted_iota(jnp.int32, aff.shape, 1)
    m1 = jnp.max(aff, axis=1, keepdims=True)
    i1 = jnp.min(jnp.where(aff == m1, lane, n_experts), axis=1, keepdims=True)
    masked = jnp.where(lane == i1, -jnp.inf, aff)
    m2 = jnp.max(masked, axis=1, keepdims=True)
    i2 = jnp.min(jnp.where(masked == m2, lane, n_experts), axis=1, keepdims=True)
    idx_ref[...] = jnp.concatenate([i1, i2], axis=1)


def kernel(hidden_states, W):
    S, B, H = hidden_states.shape
    E, _ = W.shape
    T = S * B
    x = hidden_states.reshape(T, H)
    tb = TOKEN_BLOCK
    grid = (T // tb,)

    logits, aff, idx = pl.pallas_call(
        functools.partial(_router_block_kernel, n_experts=E),
        grid=grid,
        in_specs=[
            pl.BlockSpec((tb, H), lambda i: (i, 0)),
            pl.BlockSpec((E, H), lambda i: (0, 0)),
        ],
        out_specs=[
            pl.BlockSpec((tb, E), lambda i: (i, 0)),
            pl.BlockSpec((tb, E), lambda i: (i, 0)),
            pl.BlockSpec((tb, 2), lambda i: (i, 0)),
        ],
        out_shape=[
            jax.ShapeDtypeStruct((T, E), jnp.float32),
            jax.ShapeDtypeStruct((T, E), jnp.float32),
            jax.ShapeDtypeStruct((T, 2), jnp.int32),
        ],
        compiler_params=pltpu.CompilerParams(
            dimension_semantics=("parallel",),
        ),
    )(x, W)
    return logits, aff, idx
